# baseline (speedup 1.0000x reference)
import functools

import jax
import jax.numpy as jnp
from jax import lax
from jax.experimental import pallas as pl
from jax.experimental.pallas import tpu as pltpu
from jax.experimental.pallas import tpu_sc as plsc

VOCAB = 1000000
DIM = 64
L = 16384
NC = 2
NS = 16
NW = NC * NS
B_PER_W = L // NW
LANES = 16
CHUNK = 128
NCHUNK = B_PER_W // CHUNK

_mesh = plsc.VectorSubcoreMesh(core_axis_name="c", subcore_axis_name="s")


@functools.partial(
    pl.kernel,
    mesh=_mesh,
    out_type=jax.ShapeDtypeStruct((L, DIM), jnp.float32),
    scratch_types=[
        pltpu.VMEM((NCHUNK, CHUNK), jnp.int32),
        pltpu.VMEM((NCHUNK, CHUNK), jnp.int32),
        pltpu.VMEM((2, CHUNK, 2 * DIM), jnp.float32),
        pltpu.VMEM((B_PER_W, DIM), jnp.float32),
        pltpu.SemaphoreType.DMA,
        pltpu.SemaphoreType.DMA,
    ],
)
def _gather_rows(idx_hbm, table2_hbm, out_hbm, pair_v, par_v, wide_v, rows_v,
                 sem0, sem1):
    wid = lax.axis_index("s") * NC + lax.axis_index("c")
    base = wid * B_PER_W
    for k in range(NCHUNK):
        pltpu.sync_copy(idx_hbm.at[pl.ds(base + k * CHUNK, CHUNK)],
                        pair_v.at[k])

    def to_pairs(i, _):
        k = i // (CHUNK // LANES)
        j = i % (CHUNK // LANES)
        raw = pair_v[k, pl.ds(j * LANES, LANES)]
        pair_v[k, pl.ds(j * LANES, LANES)] = raw >> 1
        par_v[k, pl.ds(j * LANES, LANES)] = (raw & 1) * DIM
        return _

    lax.fori_loop(0, B_PER_W // LANES, to_pairs, 0)

    sems = (sem0, sem1)
    copies = [None, None]
    copies[0] = pltpu.async_copy(
        table2_hbm.at[pair_v.at[0]], wide_v.at[0], sems[0])
    for k in range(NCHUNK):
        b = k % 2
        if k + 1 < NCHUNK:
            copies[1 - b] = pltpu.async_copy(
                table2_hbm.at[pair_v.at[k + 1]], wide_v.at[1 - b],
                sems[1 - b])
        copies[b].wait()

        def select_block(nb, _):
            par16 = par_v[k, pl.ds(nb * LANES, LANES)]
            for j in range(LANES):
                row = nb * LANES + j
                src = par16[j]
                for c in range(DIM // LANES):
                    rows_v[k * CHUNK + row, pl.ds(c * LANES, LANES)] = (
                        wide_v[b, row, pl.ds(src + c * LANES, LANES)])
            return _

        lax.fori_loop(0, CHUNK // LANES, select_block, 0)

    pltpu.sync_copy(rows_v, out_hbm.at[pl.ds(base, B_PER_W)])


def kernel(indices, table):
    table2 = table.reshape(VOCAB // 2, 2 * DIM)
    out = _gather_rows(indices, table2)
    return out.reshape(L, 1, DIM)


# speedup vs baseline: 1.7194x; 1.7194x over previous
import functools

import jax
import jax.numpy as jnp
from jax import lax
from jax.experimental import pallas as pl
from jax.experimental.pallas import tpu as pltpu
from jax.experimental.pallas import tpu_sc as plsc

VOCAB = 1000000
DIM = 64
L = 16384
NC = 2
NS = 16
NW = NC * NS
B_PER_W = L // NW
LANES = 16
NBLK = B_PER_W // LANES

_mesh = plsc.VectorSubcoreMesh(core_axis_name="c", subcore_axis_name="s")


@functools.partial(
    pl.kernel,
    mesh=_mesh,
    out_type=jax.ShapeDtypeStruct((L, DIM), jnp.float32),
    scratch_types=[
        pltpu.VMEM((B_PER_W,), jnp.int32),
        pltpu.VMEM((B_PER_W, DIM), jnp.float32),
        pltpu.SemaphoreType.DMA,
    ],
)
def _gather_rows(idx_hbm, table_hbm, out_hbm, idx_v, rows_v, sem0):
    wid = lax.axis_index("s") * NC + lax.axis_index("c")
    base = wid * B_PER_W
    pltpu.sync_copy(idx_hbm.at[pl.ds(base, B_PER_W)], idx_v)

    handles = []
    for nb in range(NBLK):
        idx16 = idx_v[pl.ds(nb * LANES, LANES)]
        for j in range(LANES):
            row = nb * LANES + j
            handles.append(pltpu.async_copy(
                table_hbm.at[idx16[j]], rows_v.at[row], sem0))
    for h in handles:
        h.wait()

    pltpu.sync_copy(rows_v, out_hbm.at[pl.ds(base, B_PER_W)])


def kernel(indices, table):
    out = _gather_rows(indices, table)
    return out.reshape(L, 1, DIM)
